# Initial kernel scaffold; baseline (speedup 1.0000x reference)
import jax
import jax.numpy as jnp
from jax.experimental import pallas as pl


def kernel(current_time, mgn_h, delta_t, X_obs, M_obs, i_obs, update, W_ih, W_hh, b_ih, b_hh):
    raise NotImplementedError("write your pallas kernel here")


# speedup vs baseline: 1.0836x; 1.0836x over previous
import functools

import jax
import jax.numpy as jnp
from jax import lax
from jax.experimental import pallas as pl
from jax.experimental.pallas import tpu as pltpu
from jax.experimental.pallas import tpu_sc as plsc

NC = 2
NS = 16
NW = NC * NS
CHUNK = 128


def _sc_mesh():
    return plsc.VectorSubcoreMesh(core_axis_name="c", subcore_axis_name="s")


@functools.lru_cache(maxsize=None)
def _make_gather(B, D):
    b_per_w = B // NW
    n_ch = b_per_w // CHUNK

    @functools.partial(
        pl.kernel,
        out_type=jax.ShapeDtypeStruct((B, D), jnp.float32),
        mesh=_sc_mesh(),
        scratch_types=[
            pltpu.VMEM((n_ch, CHUNK), jnp.int32),
            pltpu.VMEM((b_per_w, D), jnp.float32),
            pltpu.SemaphoreType.DMA,
        ],
    )
    def gather_kernel(bank_hbm, idx_hbm, out_hbm, idx_v, rows_v, sem):
        wid = lax.axis_index("s") * NC + lax.axis_index("c")
        pltpu.sync_copy(idx_hbm.at[wid], idx_v)
        copies = [
            pltpu.async_copy(bank_hbm.at[idx_v.at[j]],
                             rows_v.at[pl.ds(j * CHUNK, CHUNK)], sem)
            for j in range(n_ch)
        ]
        for c in copies:
            c.wait()
        pltpu.sync_copy(rows_v, out_hbm.at[pl.ds(wid * b_per_w, b_per_w)])

    return gather_kernel


@functools.lru_cache(maxsize=None)
def _make_scatter(B, D):
    b_per_w = B // NW
    n_ch = b_per_w // CHUNK

    @functools.partial(
        pl.kernel,
        out_type=(),
        mesh=_sc_mesh(),
        scratch_types=[
            pltpu.VMEM((n_ch, CHUNK), jnp.int32),
            pltpu.VMEM((n_ch, CHUNK), jnp.int32),
            pltpu.VMEM((b_per_w, D), jnp.float32),
            pltpu.SemaphoreType.DMA,
            pltpu.SemaphoreType.DMA,
        ],
    )
    def scatter_kernel(bank_ref, hnew_hbm, dst_hbm, src_hbm,
                       didx_v, sidx_v, rows_v, gsem, ssem):
        wid = lax.axis_index("s") * NC + lax.axis_index("c")
        pltpu.sync_copy(dst_hbm.at[wid], didx_v)
        pltpu.sync_copy(src_hbm.at[wid], sidx_v)
        gcopies = [
            pltpu.async_copy(hnew_hbm.at[sidx_v.at[j]],
                             rows_v.at[pl.ds(j * CHUNK, CHUNK)], gsem)
            for j in range(n_ch)
        ]
        for c in gcopies:
            c.wait()
        scopies = [
            pltpu.async_copy(rows_v.at[pl.ds(j * CHUNK, CHUNK)],
                             bank_ref.at[didx_v.at[j]], ssem)
            for j in range(n_ch)
        ]
        for c in scopies:
            c.wait()

    return scatter_kernel


def _gru_pallas(x, h, w_ihT, w_hhT, b_ih2, b_hh2):
    B, D = x.shape
    H = h.shape[1]
    R = 2048

    def body(x_ref, h_ref, wi_ref, wh_ref, bi_ref, bh_ref, out_ref):
        xb = x_ref[...]
        hb = h_ref[...]
        gi = jnp.dot(xb, wi_ref[...], preferred_element_type=jnp.float32) + bi_ref[...]
        gh = jnp.dot(hb, wh_ref[...], preferred_element_type=jnp.float32) + bh_ref[...]
        r = jax.nn.sigmoid(gi[:, :H] + gh[:, :H])
        z = jax.nn.sigmoid(gi[:, H:2 * H] + gh[:, H:2 * H])
        n = jnp.tanh(gi[:, 2 * H:] + r * gh[:, 2 * H:])
        out_ref[...] = (1.0 - z) * n + z * hb

    return pl.pallas_call(
        body,
        grid=(B // R,),
        in_specs=[
            pl.BlockSpec((R, D), lambda i: (i, 0)),
            pl.BlockSpec((R, H), lambda i: (i, 0)),
            pl.BlockSpec((D, 3 * H), lambda i: (0, 0)),
            pl.BlockSpec((H, 3 * H), lambda i: (0, 0)),
            pl.BlockSpec((1, 3 * H), lambda i: (0, 0)),
            pl.BlockSpec((1, 3 * H), lambda i: (0, 0)),
        ],
        out_specs=pl.BlockSpec((R, H), lambda i: (i, 0)),
        out_shape=jax.ShapeDtypeStruct((B, H), jnp.float32),
    )(x, h, w_ihT, w_hhT, b_ih2, b_hh2)


def kernel(current_time, mgn_h, delta_t, X_obs, M_obs, i_obs, update,
           W_ih, W_hh, b_ih, b_hh):
    N, H = mgn_h.shape
    B, D = X_obs.shape
    ii = i_obs.astype(jnp.int32)
    iota = jnp.arange(B, dtype=jnp.int32)
    win = jnp.zeros((N,), jnp.int32).at[ii].max(iota)
    src = jnp.take(win, ii)
    n_ch = (B // NW) // CHUNK
    idx2 = ii.reshape(NW, n_ch, CHUNK)
    src2 = src.reshape(NW, n_ch, CHUNK)

    h_obs = _make_gather(B, H)(mgn_h, idx2)
    h_new = _gru_pallas(X_obs, h_obs, W_ih.T, W_hh.T,
                        b_ih.reshape(1, -1), b_hh.reshape(1, -1))
    bank = jax.new_ref(mgn_h)
    _make_scatter(B, H)(bank, h_new, idx2, src2)
    return bank[...]


# speedup vs baseline: 1.5232x; 1.4057x over previous
import dataclasses
import functools

import jax
import jax.numpy as jnp
from jax import lax
from jax.experimental import pallas as pl
from jax.experimental.pallas import tpu as pltpu
from jax.experimental.pallas import tpu_sc as plsc

NC = 2
NS = 16
NW = NC * NS
CHUNK = 128
L = 16

_INT_MIN = -2147483648


def _sc_mesh():
    return plsc.VectorSubcoreMesh(core_axis_name="c", subcore_axis_name="s")


def _sc_cp():
    cp = pltpu.CompilerParams()
    if "needs_layout_passes" in pltpu.CompilerParams.__dataclass_fields__:
        cp = dataclasses.replace(cp, needs_layout_passes=False)
    return cp


def _lane0(v):
    iota = lax.iota(jnp.int32, L)
    return lax.reduce_max(
        jnp.where(iota == 0, v, jnp.full((L,), _INT_MIN, jnp.int32)), (0,))


@functools.lru_cache(maxsize=None)
def _make_gather(B, D):
    b_per_w = B // NW
    n_ch = b_per_w // CHUNK

    @functools.partial(
        pl.kernel,
        out_type=jax.ShapeDtypeStruct((B, D), jnp.float32),
        mesh=_sc_mesh(),
        scratch_types=[
            pltpu.VMEM((n_ch, CHUNK), jnp.int32),
            pltpu.VMEM((b_per_w, D), jnp.float32),
            pltpu.SemaphoreType.DMA,
        ],
    )
    def gather_kernel(bank_hbm, idx_hbm, out_hbm, idx_v, rows_v, sem):
        wid = lax.axis_index("s") * NC + lax.axis_index("c")
        pltpu.sync_copy(idx_hbm.at[wid], idx_v)
        copies = [
            pltpu.async_copy(bank_hbm.at[idx_v.at[j]],
                             rows_v.at[pl.ds(j * CHUNK, CHUNK)], sem)
            for j in range(n_ch)
        ]
        for c in copies:
            c.wait()
        pltpu.sync_copy(rows_v, out_hbm.at[pl.ds(wid * b_per_w, b_per_w)])

    return gather_kernel


@functools.lru_cache(maxsize=None)
def _make_winner_scatter(N, B, D):
    rows_per_w = N // NW
    win_sz = ((rows_per_w + L - 1) // L) * L
    n_win_vecs = win_sz // L
    max_ch = (rows_per_w + CHUNK - 1) // CHUNK + 1
    list_sz = max_ch * CHUNK
    PIPE = 4

    @functools.partial(
        pl.kernel,
        out_type=(),
        mesh=_sc_mesh(),
        compiler_params=_sc_cp(),
        scratch_types=[
            pltpu.VMEM((B,), jnp.int32),
            pltpu.VMEM((win_sz,), jnp.int32),
            pltpu.VMEM((list_sz,), jnp.int32),
            pltpu.VMEM((list_sz,), jnp.int32),
            pltpu.VMEM((max_ch, CHUNK), jnp.int32),
            pltpu.VMEM((max_ch, CHUNK), jnp.int32),
            pltpu.VMEM((PIPE * CHUNK, D), jnp.float32),
            pltpu.SemaphoreType.DMA,
            pltpu.SemaphoreType.DMA,
        ],
    )
    def scatter_kernel(bank_ref, hnew_hbm, idx_hbm,
                       idx_all, win_v, dstf, srcf, dst2, src2, rows_v,
                       gsem, ssem):
        wid = lax.axis_index("s") * NC + lax.axis_index("c")
        lo = wid * rows_per_w
        hi = lo + rows_per_w
        iota = lax.iota(jnp.int32, L)
        pltpu.sync_copy(idx_hbm, idx_all)

        neg1 = jnp.full((L,), -1, jnp.int32)

        @pl.loop(0, n_win_vecs)
        def _(j):
            win_v[pl.ds(j * L, L)] = neg1

        @pl.loop(0, B // L)
        def _(i):
            v = idx_all[pl.ds(i * L, L)]
            b = iota + i * L
            inr = (v >= lo) & (v < hi)
            _, lastm = plsc.scan_count(v)
            m = inr & lastm
            loc = lax.min(lax.max(v - lo, jnp.zeros((L,), jnp.int32)),
                          jnp.full((L,), win_sz - 1, jnp.int32))
            plsc.store_scatter(win_v.at[:], [loc], b, mask=m)

        def compact(j, off):
            wv = win_v[pl.ds(j * L, L)]
            m = wv >= 0
            rows = iota + (j * L + lo)
            plsc.store_compressed(dstf.at[pl.ds(off, L)], rows, mask=m)
            plsc.store_compressed(srcf.at[pl.ds(off, L)], wv, mask=m)
            return off + lax.reduce_max(plsc.all_reduce_population_count(m), (0,))

        cnt = lax.fori_loop(0, n_win_vecs, compact, jnp.int32(0))
        n_chunks = (cnt + CHUNK - 1) // CHUNK

        @pl.when(cnt > 0)
        def _():
            d0 = _lane0(dstf[pl.ds(0, L)])
            s0 = _lane0(srcf[pl.ds(0, L)])

            @pl.loop(0, (n_chunks * CHUNK - (cnt // L) * L) // L)
            def _(t):
                base = (cnt // L) * L + t * L
                keep = iota + base < cnt
                dv = dstf[pl.ds(base, L)]
                sv = srcf[pl.ds(base, L)]
                dstf[pl.ds(base, L)] = jnp.where(keep, dv, jnp.full((L,), 0, jnp.int32) + d0)
                srcf[pl.ds(base, L)] = jnp.where(keep, sv, jnp.full((L,), 0, jnp.int32) + s0)

            @pl.loop(0, n_chunks * (CHUNK // L))
            def _(k):
                c = k // (CHUNK // L)
                kk = lax.rem(k, jnp.int32(CHUNK // L))
                dst2[c, pl.ds(kk * L, L)] = dstf[pl.ds(k * L, L)]
                src2[c, pl.ds(kk * L, L)] = srcf[pl.ds(k * L, L)]

            n_rounds = (n_chunks + PIPE - 1) // PIPE

            @pl.loop(0, n_rounds)
            def _(r):
                def each(k, fn):
                    c = r * PIPE + k

                    @pl.when(c < n_chunks)
                    def _():
                        fn(c, k)

                def g_args(c, k):
                    return (hnew_hbm.at[src2.at[c]],
                            rows_v.at[pl.ds(k * CHUNK, CHUNK)], gsem)

                def s_args(c, k):
                    return (rows_v.at[pl.ds(k * CHUNK, CHUNK)],
                            bank_ref.at[dst2.at[c]], ssem)

                for k in range(PIPE):
                    each(k, lambda c, kk: pltpu.async_copy(*g_args(c, kk)))
                for k in range(PIPE):
                    each(k, lambda c, kk: pltpu.make_async_copy(*g_args(c, kk)).wait())
                for k in range(PIPE):
                    each(k, lambda c, kk: pltpu.async_copy(*s_args(c, kk)))
                for k in range(PIPE):
                    each(k, lambda c, kk: pltpu.make_async_copy(*s_args(c, kk)).wait())

    return scatter_kernel


def _gru_pallas(x, h, w_ihT, w_hhT, b_ih2, b_hh2):
    B, D = x.shape
    H = h.shape[1]
    R = 2048

    def body(x_ref, h_ref, wi_ref, wh_ref, bi_ref, bh_ref, out_ref):
        xb = x_ref[...]
        hb = h_ref[...]
        gi = jnp.dot(xb, wi_ref[...], preferred_element_type=jnp.float32) + bi_ref[...]
        gh = jnp.dot(hb, wh_ref[...], preferred_element_type=jnp.float32) + bh_ref[...]
        r = jax.nn.sigmoid(gi[:, :H] + gh[:, :H])
        z = jax.nn.sigmoid(gi[:, H:2 * H] + gh[:, H:2 * H])
        n = jnp.tanh(gi[:, 2 * H:] + r * gh[:, 2 * H:])
        out_ref[...] = (1.0 - z) * n + z * hb

    return pl.pallas_call(
        body,
        grid=(B // R,),
        in_specs=[
            pl.BlockSpec((R, D), lambda i: (i, 0)),
            pl.BlockSpec((R, H), lambda i: (i, 0)),
            pl.BlockSpec((D, 3 * H), lambda i: (0, 0)),
            pl.BlockSpec((H, 3 * H), lambda i: (0, 0)),
            pl.BlockSpec((1, 3 * H), lambda i: (0, 0)),
            pl.BlockSpec((1, 3 * H), lambda i: (0, 0)),
        ],
        out_specs=pl.BlockSpec((R, H), lambda i: (i, 0)),
        out_shape=jax.ShapeDtypeStruct((B, H), jnp.float32),
    )(x, h, w_ihT, w_hhT, b_ih2, b_hh2)


def kernel(current_time, mgn_h, delta_t, X_obs, M_obs, i_obs, update,
           W_ih, W_hh, b_ih, b_hh):
    N, H = mgn_h.shape
    B, D = X_obs.shape
    ii = i_obs.astype(jnp.int32)
    n_ch = (B // NW) // CHUNK
    idx2 = ii.reshape(NW, n_ch, CHUNK)

    h_obs = _make_gather(B, H)(mgn_h, idx2)
    h_new = _gru_pallas(X_obs, h_obs, W_ih.T, W_hh.T,
                        b_ih.reshape(1, -1), b_hh.reshape(1, -1))
    bank = jax.new_ref(mgn_h)
    _make_winner_scatter(N, B, H)(bank, h_new, ii)
    return bank[...]


# speedup vs baseline: 1.5275x; 1.0028x over previous
import dataclasses
import functools

import jax
import jax.numpy as jnp
from jax import lax
from jax.experimental import pallas as pl
from jax.experimental.pallas import tpu as pltpu
from jax.experimental.pallas import tpu_sc as plsc

NC = 2
NS = 16
NW = NC * NS
CHUNK = 128
L = 16
PIPE = 4

_INT_MIN = -2147483648


def _sc_mesh():
    return plsc.VectorSubcoreMesh(core_axis_name="c", subcore_axis_name="s")


def _sc_cp():
    cp = pltpu.CompilerParams()
    if "needs_layout_passes" in pltpu.CompilerParams.__dataclass_fields__:
        cp = dataclasses.replace(cp, needs_layout_passes=False)
    return cp


def _lane0(v):
    iota = lax.iota(jnp.int32, L)
    return lax.reduce_max(
        jnp.where(iota == 0, v, jnp.full((L,), _INT_MIN, jnp.int32)), (0,))


@functools.lru_cache(maxsize=None)
def _make_gather_plan(N, B, D):
    b_per_w = B // NW
    n_gch = b_per_w // CHUNK
    rows_per_w = N // NW
    win_sz = ((rows_per_w + L - 1) // L) * L
    n_win_vecs = win_sz // L
    max_ch = (rows_per_w + CHUNK - 1) // CHUNK + 1
    list_sz = max_ch * CHUNK

    @functools.partial(
        pl.kernel,
        out_type=[jax.ShapeDtypeStruct((B, D), jnp.float32),
                  jax.ShapeDtypeStruct((NW, max_ch, CHUNK), jnp.int32),
                  jax.ShapeDtypeStruct((NW, max_ch, CHUNK), jnp.int32),
                  jax.ShapeDtypeStruct((NW, L), jnp.int32)],
        mesh=_sc_mesh(),
        compiler_params=_sc_cp(),
        scratch_types=[
            pltpu.VMEM((B,), jnp.int32),
            pltpu.VMEM((b_per_w, D), jnp.float32),
            pltpu.VMEM((win_sz,), jnp.int32),
            pltpu.VMEM((list_sz,), jnp.int32),
            pltpu.VMEM((list_sz,), jnp.int32),
            pltpu.VMEM((max_ch, CHUNK), jnp.int32),
            pltpu.VMEM((max_ch, CHUNK), jnp.int32),
            pltpu.VMEM((L,), jnp.int32),
            pltpu.SemaphoreType.DMA,
        ],
    )
    def gather_plan_kernel(bank_hbm, idx_hbm, hobs_hbm, dst_hbm, src_hbm,
                           cnt_hbm, idx_all, rows_v, win_v, dstf, srcf,
                           dst2, src2, cnt_v, sem):
        wid = lax.axis_index("s") * NC + lax.axis_index("c")
        lo = wid * rows_per_w
        hi = lo + rows_per_w
        iota = lax.iota(jnp.int32, L)
        pltpu.sync_copy(idx_hbm, idx_all)

        def g_args(j):
            return (bank_hbm.at[idx_all.at[pl.ds(wid * b_per_w + j * CHUNK, CHUNK)]],
                    rows_v.at[pl.ds(j * CHUNK, CHUNK)], sem)

        for j in range(n_gch):
            pltpu.async_copy(*g_args(j))

        neg1 = jnp.full((L,), -1, jnp.int32)

        @pl.loop(0, n_win_vecs)
        def _(j):
            win_v[pl.ds(j * L, L)] = neg1

        @pl.loop(0, B // (2 * L))
        def _(i):
            for u in range(2):
                base = (2 * i + u) * L
                v = idx_all[pl.ds(base, L)]
                b = iota + base
                inr = (v >= lo) & (v < hi)
                _, lastm = plsc.scan_count(v)
                m = inr & lastm
                loc = lax.min(lax.max(v - lo, jnp.zeros((L,), jnp.int32)),
                              jnp.full((L,), win_sz - 1, jnp.int32))
                plsc.store_scatter(win_v.at[:], [loc], b, mask=m)

        def compact(j, off):
            wv = win_v[pl.ds(j * L, L)]
            m = wv >= 0
            rows = iota + (j * L + lo)
            plsc.store_compressed(dstf.at[pl.ds(off, L)], rows, mask=m)
            plsc.store_compressed(srcf.at[pl.ds(off, L)], wv, mask=m)
            return off + lax.reduce_max(plsc.all_reduce_population_count(m), (0,))

        cnt = lax.fori_loop(0, n_win_vecs, compact, jnp.int32(0))
        n_chunks = (cnt + CHUNK - 1) // CHUNK
        cnt_v[...] = jnp.full((L,), 0, jnp.int32) + n_chunks
        pltpu.sync_copy(cnt_v, cnt_hbm.at[wid])

        @pl.when(cnt > 0)
        def _():
            d0 = _lane0(dstf[pl.ds(0, L)])
            s0 = _lane0(srcf[pl.ds(0, L)])

            @pl.loop(0, (n_chunks * CHUNK - (cnt // L) * L) // L)
            def _(t):
                base = (cnt // L) * L + t * L
                keep = iota + base < cnt
                dv = dstf[pl.ds(base, L)]
                sv = srcf[pl.ds(base, L)]
                dstf[pl.ds(base, L)] = jnp.where(keep, dv, jnp.full((L,), 0, jnp.int32) + d0)
                srcf[pl.ds(base, L)] = jnp.where(keep, sv, jnp.full((L,), 0, jnp.int32) + s0)

            @pl.loop(0, n_chunks * (CHUNK // L))
            def _(k):
                c = k // (CHUNK // L)
                kk = lax.rem(k, jnp.int32(CHUNK // L))
                dst2[c, pl.ds(kk * L, L)] = dstf[pl.ds(k * L, L)]
                src2[c, pl.ds(kk * L, L)] = srcf[pl.ds(k * L, L)]

            pltpu.sync_copy(dst2, dst_hbm.at[wid])
            pltpu.sync_copy(src2, src_hbm.at[wid])

        for j in range(n_gch):
            pltpu.make_async_copy(*g_args(j)).wait()
        pltpu.sync_copy(rows_v, hobs_hbm.at[pl.ds(wid * b_per_w, b_per_w)])

    return gather_plan_kernel


@functools.lru_cache(maxsize=None)
def _make_scatter(N, B, D):
    rows_per_w = N // NW
    max_ch = (rows_per_w + CHUNK - 1) // CHUNK + 1

    @functools.partial(
        pl.kernel,
        out_type=(),
        mesh=_sc_mesh(),
        compiler_params=_sc_cp(),
        scratch_types=[
            pltpu.VMEM((max_ch, CHUNK), jnp.int32),
            pltpu.VMEM((max_ch, CHUNK), jnp.int32),
            pltpu.VMEM((L,), jnp.int32),
            pltpu.VMEM((PIPE * CHUNK, D), jnp.float32),
            pltpu.SemaphoreType.DMA,
            pltpu.SemaphoreType.DMA,
        ],
    )
    def scatter_kernel(bank_ref, hnew_hbm, dst_hbm, src_hbm, cnt_hbm,
                       dst2, src2, cnt_v, rows_v, gsem, ssem):
        wid = lax.axis_index("s") * NC + lax.axis_index("c")
        pltpu.sync_copy(cnt_hbm.at[wid], cnt_v)
        n_chunks = _lane0(cnt_v[...])

        @pl.when(n_chunks > 0)
        def _():
            pltpu.sync_copy(dst_hbm.at[wid], dst2)
            pltpu.sync_copy(src_hbm.at[wid], src2)
            n_rounds = (n_chunks + PIPE - 1) // PIPE

            @pl.loop(0, n_rounds)
            def _(r):
                def each(k, fn):
                    c = r * PIPE + k

                    @pl.when(c < n_chunks)
                    def _():
                        fn(c, k)

                def g_args(c, k):
                    return (hnew_hbm.at[src2.at[c]],
                            rows_v.at[pl.ds(k * CHUNK, CHUNK)], gsem)

                def s_args(c, k):
                    return (rows_v.at[pl.ds(k * CHUNK, CHUNK)],
                            bank_ref.at[dst2.at[c]], ssem)

                for k in range(PIPE):
                    each(k, lambda c, kk: pltpu.async_copy(*g_args(c, kk)))
                for k in range(PIPE):
                    each(k, lambda c, kk: pltpu.make_async_copy(*g_args(c, kk)).wait())
                for k in range(PIPE):
                    each(k, lambda c, kk: pltpu.async_copy(*s_args(c, kk)))
                for k in range(PIPE):
                    each(k, lambda c, kk: pltpu.make_async_copy(*s_args(c, kk)).wait())

    return scatter_kernel


def _gru_pallas(x, h, w_ihT, w_hhT, b_ih2, b_hh2):
    B, D = x.shape
    H = h.shape[1]
    R = 2048

    def body(x_ref, h_ref, wi_ref, wh_ref, bi_ref, bh_ref, out_ref):
        xb = x_ref[...]
        hb = h_ref[...]
        gi = jnp.dot(xb, wi_ref[...], preferred_element_type=jnp.float32) + bi_ref[...]
        gh = jnp.dot(hb, wh_ref[...], preferred_element_type=jnp.float32) + bh_ref[...]
        r = jax.nn.sigmoid(gi[:, :H] + gh[:, :H])
        z = jax.nn.sigmoid(gi[:, H:2 * H] + gh[:, H:2 * H])
        n = jnp.tanh(gi[:, 2 * H:] + r * gh[:, 2 * H:])
        out_ref[...] = (1.0 - z) * n + z * hb

    return pl.pallas_call(
        body,
        grid=(B // R,),
        in_specs=[
            pl.BlockSpec((R, D), lambda i: (i, 0)),
            pl.BlockSpec((R, H), lambda i: (i, 0)),
            pl.BlockSpec((D, 3 * H), lambda i: (0, 0)),
            pl.BlockSpec((H, 3 * H), lambda i: (0, 0)),
            pl.BlockSpec((1, 3 * H), lambda i: (0, 0)),
            pl.BlockSpec((1, 3 * H), lambda i: (0, 0)),
        ],
        out_specs=pl.BlockSpec((R, H), lambda i: (i, 0)),
        out_shape=jax.ShapeDtypeStruct((B, H), jnp.float32),
    )(x, h, w_ihT, w_hhT, b_ih2, b_hh2)


def kernel(current_time, mgn_h, delta_t, X_obs, M_obs, i_obs, update,
           W_ih, W_hh, b_ih, b_hh):
    N, H = mgn_h.shape
    B, D = X_obs.shape
    ii = i_obs.astype(jnp.int32)

    h_obs, dst_plan, src_plan, cnt_plan = _make_gather_plan(N, B, H)(mgn_h, ii)
    h_new = _gru_pallas(X_obs, h_obs, W_ih.T, W_hh.T,
                        b_ih.reshape(1, -1), b_hh.reshape(1, -1))
    bank = jax.new_ref(mgn_h)
    _make_scatter(N, B, H)(bank, h_new, dst_plan, src_plan, cnt_plan)
    return bank[...]


# speedup vs baseline: 1.5572x; 1.0194x over previous
import dataclasses
import functools

import jax
import jax.numpy as jnp
from jax import lax
from jax.experimental import pallas as pl
from jax.experimental.pallas import tpu as pltpu
from jax.experimental.pallas import tpu_sc as plsc

NC = 2
NS = 16
NW = NC * NS
CHUNK = 128
L = 16
PIPE = 3
NSLOT = 2 * PIPE

_INT_MIN = -2147483648


def _sc_mesh():
    return plsc.VectorSubcoreMesh(core_axis_name="c", subcore_axis_name="s")


def _sc_cp():
    cp = pltpu.CompilerParams()
    if "needs_layout_passes" in pltpu.CompilerParams.__dataclass_fields__:
        cp = dataclasses.replace(cp, needs_layout_passes=False)
    return cp


def _lane0(v):
    iota = lax.iota(jnp.int32, L)
    return lax.reduce_max(
        jnp.where(iota == 0, v, jnp.full((L,), _INT_MIN, jnp.int32)), (0,))


@functools.lru_cache(maxsize=None)
def _make_gather_plan(N, B, D):
    b_per_w = B // NW
    n_gch = b_per_w // CHUNK
    rows_per_w = N // NW
    win_sz = ((rows_per_w + L - 1) // L) * L
    n_win_vecs = win_sz // L
    max_ch = (rows_per_w + CHUNK - 1) // CHUNK + 1
    list_sz = max_ch * CHUNK

    @functools.partial(
        pl.kernel,
        out_type=[jax.ShapeDtypeStruct((B, D), jnp.float32),
                  jax.ShapeDtypeStruct((NW, max_ch, CHUNK), jnp.int32),
                  jax.ShapeDtypeStruct((NW, max_ch, CHUNK), jnp.int32),
                  jax.ShapeDtypeStruct((NW, L), jnp.int32)],
        mesh=_sc_mesh(),
        compiler_params=_sc_cp(),
        scratch_types=[
            pltpu.VMEM((B,), jnp.int32),
            pltpu.VMEM((b_per_w, D), jnp.float32),
            pltpu.VMEM((win_sz,), jnp.int32),
            pltpu.VMEM((list_sz,), jnp.int32),
            pltpu.VMEM((list_sz,), jnp.int32),
            pltpu.VMEM((max_ch, CHUNK), jnp.int32),
            pltpu.VMEM((max_ch, CHUNK), jnp.int32),
            pltpu.VMEM((L,), jnp.int32),
            pltpu.SemaphoreType.DMA,
        ],
    )
    def gather_plan_kernel(bank_hbm, idx_hbm, hobs_hbm, dst_hbm, src_hbm,
                           cnt_hbm, idx_all, rows_v, win_v, dstf, srcf,
                           dst2, src2, cnt_v, sem):
        wid = lax.axis_index("s") * NC + lax.axis_index("c")
        lo = wid * rows_per_w
        hi = lo + rows_per_w
        iota = lax.iota(jnp.int32, L)
        pltpu.sync_copy(idx_hbm, idx_all)

        def g_args(j):
            return (bank_hbm.at[idx_all.at[pl.ds(wid * b_per_w + j * CHUNK, CHUNK)]],
                    rows_v.at[pl.ds(j * CHUNK, CHUNK)], sem)

        for j in range(n_gch):
            pltpu.async_copy(*g_args(j))

        neg1 = jnp.full((L,), -1, jnp.int32)

        @pl.loop(0, n_win_vecs)
        def _(j):
            win_v[pl.ds(j * L, L)] = neg1

        @pl.loop(0, B // (2 * L))
        def _(i):
            for u in range(2):
                base = (2 * i + u) * L
                v = idx_all[pl.ds(base, L)]
                b = iota + base
                inr = (v >= lo) & (v < hi)
                _, lastm = plsc.scan_count(v)
                m = inr & lastm
                loc = lax.min(lax.max(v - lo, jnp.zeros((L,), jnp.int32)),
                              jnp.full((L,), win_sz - 1, jnp.int32))
                plsc.store_scatter(win_v.at[:], [loc], b, mask=m)

        def compact(j, off):
            wv = win_v[pl.ds(j * L, L)]
            m = wv >= 0
            rows = iota + (j * L + lo)
            plsc.store_compressed(dstf.at[pl.ds(off, L)], rows, mask=m)
            plsc.store_compressed(srcf.at[pl.ds(off, L)], wv, mask=m)
            return off + lax.reduce_max(plsc.all_reduce_population_count(m), (0,))

        cnt = lax.fori_loop(0, n_win_vecs, compact, jnp.int32(0))
        n_chunks = (cnt + CHUNK - 1) // CHUNK
        cnt_v[...] = jnp.full((L,), 0, jnp.int32) + n_chunks
        pltpu.sync_copy(cnt_v, cnt_hbm.at[wid])

        @pl.when(cnt > 0)
        def _():
            d0 = _lane0(dstf[pl.ds(0, L)])
            s0 = _lane0(srcf[pl.ds(0, L)])

            @pl.loop(0, (n_chunks * CHUNK - (cnt // L) * L) // L)
            def _(t):
                base = (cnt // L) * L + t * L
                keep = iota + base < cnt
                dv = dstf[pl.ds(base, L)]
                sv = srcf[pl.ds(base, L)]
                dstf[pl.ds(base, L)] = jnp.where(keep, dv, jnp.full((L,), 0, jnp.int32) + d0)
                srcf[pl.ds(base, L)] = jnp.where(keep, sv, jnp.full((L,), 0, jnp.int32) + s0)

            @pl.loop(0, n_chunks * (CHUNK // L))
            def _(k):
                c = k // (CHUNK // L)
                kk = lax.rem(k, jnp.int32(CHUNK // L))
                dst2[c, pl.ds(kk * L, L)] = dstf[pl.ds(k * L, L)]
                src2[c, pl.ds(kk * L, L)] = srcf[pl.ds(k * L, L)]

            pltpu.sync_copy(dst2, dst_hbm.at[wid])
            pltpu.sync_copy(src2, src_hbm.at[wid])

        for j in range(n_gch):
            pltpu.make_async_copy(*g_args(j)).wait()
        pltpu.sync_copy(rows_v, hobs_hbm.at[pl.ds(wid * b_per_w, b_per_w)])

    return gather_plan_kernel


@functools.lru_cache(maxsize=None)
def _make_scatter(N, B, D):
    rows_per_w = N // NW
    max_ch = (rows_per_w + CHUNK - 1) // CHUNK + 1

    @functools.partial(
        pl.kernel,
        out_type=(),
        mesh=_sc_mesh(),
        compiler_params=_sc_cp(),
        scratch_types=[
            pltpu.VMEM((max_ch, CHUNK), jnp.int32),
            pltpu.VMEM((max_ch, CHUNK), jnp.int32),
            pltpu.VMEM((L,), jnp.int32),
            pltpu.VMEM((NSLOT * CHUNK, D), jnp.float32),
            pltpu.SemaphoreType.DMA,
            pltpu.SemaphoreType.DMA,
        ],
    )
    def scatter_kernel(bank_ref, hnew_hbm, dst_hbm, src_hbm, cnt_hbm,
                       dst2, src2, cnt_v, rows_v, gsem, ssem):
        wid = lax.axis_index("s") * NC + lax.axis_index("c")
        pltpu.sync_copy(cnt_hbm.at[wid], cnt_v)
        n_chunks = _lane0(cnt_v[...])

        @pl.when(n_chunks > 0)
        def _():
            pltpu.sync_copy(dst_hbm.at[wid], dst2)
            pltpu.sync_copy(src_hbm.at[wid], src2)

            def g_args(c):
                slot = lax.rem(c, jnp.int32(NSLOT))
                return (hnew_hbm.at[src2.at[c]],
                        rows_v.at[pl.ds(slot * CHUNK, CHUNK)], gsem)

            def s_args(c):
                slot = lax.rem(c, jnp.int32(NSLOT))
                return (rows_v.at[pl.ds(slot * CHUNK, CHUNK)],
                        bank_ref.at[dst2.at[c]], ssem)

            def when_chunk(pred, fn):
                @pl.when(pred)
                def _():
                    fn()

            for c0 in range(PIPE):
                when_chunk(c0 < n_chunks,
                           lambda c=c0: pltpu.async_copy(*g_args(jnp.int32(c))))

            @pl.loop(0, n_chunks)
            def _(c):
                when_chunk(c >= PIPE,
                           lambda: pltpu.make_async_copy(*s_args(c - PIPE)).wait())
                pltpu.make_async_copy(*g_args(c)).wait()
                pltpu.async_copy(*s_args(c))
                when_chunk(c + PIPE < n_chunks,
                           lambda: pltpu.async_copy(*g_args(c + PIPE)))

            @pl.loop(lax.max(jnp.int32(0), n_chunks - PIPE), n_chunks)
            def _(c):
                pltpu.make_async_copy(*s_args(c)).wait()

    return scatter_kernel


def _gru_pallas(x, h, w_ihT, w_hhT, b_ih2, b_hh2, bank_in):
    B, D = x.shape
    H = h.shape[1]
    N = bank_in.shape[0]
    R = 2048

    def body(x_ref, h_ref, wi_ref, wh_ref, bi_ref, bh_ref, bank_ref,
             out_ref, bank_out_ref):
        del bank_ref, bank_out_ref
        xb = x_ref[...]
        hb = h_ref[...]
        gi = jnp.dot(xb, wi_ref[...], preferred_element_type=jnp.float32) + bi_ref[...]
        gh = jnp.dot(hb, wh_ref[...], preferred_element_type=jnp.float32) + bh_ref[...]
        r = jax.nn.sigmoid(gi[:, :H] + gh[:, :H])
        z = jax.nn.sigmoid(gi[:, H:2 * H] + gh[:, H:2 * H])
        n = jnp.tanh(gi[:, 2 * H:] + r * gh[:, 2 * H:])
        out_ref[...] = (1.0 - z) * n + z * hb

    return pl.pallas_call(
        body,
        grid=(B // R,),
        in_specs=[
            pl.BlockSpec((R, D), lambda i: (i, 0)),
            pl.BlockSpec((R, H), lambda i: (i, 0)),
            pl.BlockSpec((D, 3 * H), lambda i: (0, 0)),
            pl.BlockSpec((H, 3 * H), lambda i: (0, 0)),
            pl.BlockSpec((1, 3 * H), lambda i: (0, 0)),
            pl.BlockSpec((1, 3 * H), lambda i: (0, 0)),
            pl.BlockSpec(memory_space=pl.ANY),
        ],
        out_specs=[pl.BlockSpec((R, H), lambda i: (i, 0)),
                   pl.BlockSpec(memory_space=pl.ANY)],
        out_shape=[jax.ShapeDtypeStruct((B, H), jnp.float32),
                   jax.ShapeDtypeStruct((N, H), jnp.float32)],
        input_output_aliases={6: 1},
    )(x, h, w_ihT, w_hhT, b_ih2, b_hh2, bank_in)


def kernel(current_time, mgn_h, delta_t, X_obs, M_obs, i_obs, update,
           W_ih, W_hh, b_ih, b_hh):
    N, H = mgn_h.shape
    B, D = X_obs.shape
    ii = i_obs.astype(jnp.int32)

    h_obs, dst_plan, src_plan, cnt_plan = _make_gather_plan(N, B, H)(mgn_h, ii)
    h_new, bank_out = _gru_pallas(X_obs, h_obs, W_ih.T, W_hh.T,
                                  b_ih.reshape(1, -1), b_hh.reshape(1, -1),
                                  mgn_h)
    bank = jax.new_ref(bank_out)
    _make_scatter(N, B, H)(bank, h_new, dst_plan, src_plan, cnt_plan)
    return bank[...]


# speedup vs baseline: 1.6720x; 1.0737x over previous
import dataclasses
import functools

import jax
import jax.numpy as jnp
from jax import lax
from jax.experimental import pallas as pl
from jax.experimental.pallas import tpu as pltpu
from jax.experimental.pallas import tpu_sc as plsc

NC = 2
NS = 16
NW = NC * NS
CHUNK = 128
L = 16
PIPE = 3
NSLOT = 2 * PIPE

_INT_MIN = -2147483648


def _sc_mesh():
    return plsc.VectorSubcoreMesh(core_axis_name="c", subcore_axis_name="s")


def _sc_cp():
    cp = pltpu.CompilerParams()
    if "needs_layout_passes" in pltpu.CompilerParams.__dataclass_fields__:
        cp = dataclasses.replace(cp, needs_layout_passes=False)
    return cp


def _lane0(v):
    iota = lax.iota(jnp.int32, L)
    return lax.reduce_max(
        jnp.where(iota == 0, v, jnp.full((L,), _INT_MIN, jnp.int32)), (0,))


@functools.lru_cache(maxsize=None)
def _make_gather_plan(N, B, D):
    b_per_w = B // NW
    n_gch = b_per_w // CHUNK
    rows_per_w = N // NW
    win_sz = ((rows_per_w + L - 1) // L) * L
    n_win_vecs = win_sz // L
    max_ch = (rows_per_w + CHUNK - 1) // CHUNK + 1
    list_sz = max_ch * CHUNK

    @functools.partial(
        pl.kernel,
        out_type=[jax.ShapeDtypeStruct((B, D), jnp.float32),
                  jax.ShapeDtypeStruct((NW, max_ch, CHUNK), jnp.int32),
                  jax.ShapeDtypeStruct((NW, max_ch, CHUNK), jnp.int32),
                  jax.ShapeDtypeStruct((NW, L), jnp.int32)],
        mesh=_sc_mesh(),
        compiler_params=_sc_cp(),
        scratch_types=[
            pltpu.VMEM((B,), jnp.int32),
            pltpu.VMEM((b_per_w, D), jnp.float32),
            pltpu.VMEM((win_sz,), jnp.int32),
            pltpu.VMEM((list_sz,), jnp.int32),
            pltpu.VMEM((list_sz,), jnp.int32),
            pltpu.VMEM((max_ch, CHUNK), jnp.int32),
            pltpu.VMEM((max_ch, CHUNK), jnp.int32),
            pltpu.VMEM((L,), jnp.int32),
            pltpu.SemaphoreType.DMA,
        ],
    )
    def gather_plan_kernel(bank_hbm, idx_hbm, hobs_hbm, dst_hbm, src_hbm,
                           cnt_hbm, idx_all, rows_v, win_v, dstf, srcf,
                           dst2, src2, cnt_v, sem):
        wid = lax.axis_index("s") * NC + lax.axis_index("c")
        lo = wid * rows_per_w
        hi = lo + rows_per_w
        iota = lax.iota(jnp.int32, L)
        pltpu.sync_copy(idx_hbm, idx_all)

        def g_args(j):
            return (bank_hbm.at[idx_all.at[pl.ds(wid * b_per_w + j * CHUNK, CHUNK)]],
                    rows_v.at[pl.ds(j * CHUNK, CHUNK)], sem)

        for j in range(n_gch):
            pltpu.async_copy(*g_args(j))

        neg1 = jnp.full((L,), -1, jnp.int32)

        @pl.loop(0, n_win_vecs)
        def _(j):
            win_v[pl.ds(j * L, L)] = neg1

        @pl.loop(0, B // (4 * L))
        def _(i):
            for u in range(4):
                base = (4 * i + u) * L
                v = idx_all[pl.ds(base, L)]
                b = iota + base
                m = (v >= lo) & (v < hi)
                plsc.store_scatter(win_v.at[:], [v - lo], b, mask=m)

        def compact(j, off):
            wv = win_v[pl.ds(j * L, L)]
            m = wv >= 0
            rows = iota + (j * L + lo)
            plsc.store_compressed(dstf.at[pl.ds(off, L)], rows, mask=m)
            plsc.store_compressed(srcf.at[pl.ds(off, L)], wv, mask=m)
            return off + lax.reduce_max(plsc.all_reduce_population_count(m), (0,))

        cnt = lax.fori_loop(0, n_win_vecs, compact, jnp.int32(0))
        n_chunks = (cnt + CHUNK - 1) // CHUNK
        cnt_v[...] = jnp.full((L,), 0, jnp.int32) + n_chunks
        pltpu.sync_copy(cnt_v, cnt_hbm.at[wid])

        @pl.when(cnt > 0)
        def _():
            d0 = _lane0(dstf[pl.ds(0, L)])
            s0 = _lane0(srcf[pl.ds(0, L)])

            @pl.loop(0, (n_chunks * CHUNK - (cnt // L) * L) // L)
            def _(t):
                base = (cnt // L) * L + t * L
                keep = iota + base < cnt
                dv = dstf[pl.ds(base, L)]
                sv = srcf[pl.ds(base, L)]
                dstf[pl.ds(base, L)] = jnp.where(keep, dv, jnp.full((L,), 0, jnp.int32) + d0)
                srcf[pl.ds(base, L)] = jnp.where(keep, sv, jnp.full((L,), 0, jnp.int32) + s0)

            @pl.loop(0, n_chunks * (CHUNK // L))
            def _(k):
                c = k // (CHUNK // L)
                kk = lax.rem(k, jnp.int32(CHUNK // L))
                dst2[c, pl.ds(kk * L, L)] = dstf[pl.ds(k * L, L)]
                src2[c, pl.ds(kk * L, L)] = srcf[pl.ds(k * L, L)]

            pltpu.sync_copy(dst2, dst_hbm.at[wid])
            pltpu.sync_copy(src2, src_hbm.at[wid])

        for j in range(n_gch):
            pltpu.make_async_copy(*g_args(j)).wait()
        pltpu.sync_copy(rows_v, hobs_hbm.at[pl.ds(wid * b_per_w, b_per_w)])

    return gather_plan_kernel


@functools.lru_cache(maxsize=None)
def _make_scatter(N, B, D):
    rows_per_w = N // NW
    max_ch = (rows_per_w + CHUNK - 1) // CHUNK + 1

    @functools.partial(
        pl.kernel,
        out_type=(),
        mesh=_sc_mesh(),
        compiler_params=_sc_cp(),
        scratch_types=[
            pltpu.VMEM((max_ch, CHUNK), jnp.int32),
            pltpu.VMEM((max_ch, CHUNK), jnp.int32),
            pltpu.VMEM((L,), jnp.int32),
            pltpu.VMEM((NSLOT * CHUNK, D), jnp.float32),
            pltpu.SemaphoreType.DMA,
            pltpu.SemaphoreType.DMA,
        ],
    )
    def scatter_kernel(bank_ref, hnew_hbm, dst_hbm, src_hbm, cnt_hbm,
                       dst2, src2, cnt_v, rows_v, gsem, ssem):
        wid = lax.axis_index("s") * NC + lax.axis_index("c")
        pltpu.sync_copy(cnt_hbm.at[wid], cnt_v)
        n_chunks = _lane0(cnt_v[...])

        @pl.when(n_chunks > 0)
        def _():
            pltpu.sync_copy(dst_hbm.at[wid], dst2)
            pltpu.sync_copy(src_hbm.at[wid], src2)

            def g_args(c):
                slot = lax.rem(c, jnp.int32(NSLOT))
                return (hnew_hbm.at[src2.at[c]],
                        rows_v.at[pl.ds(slot * CHUNK, CHUNK)], gsem)

            def s_args(c):
                slot = lax.rem(c, jnp.int32(NSLOT))
                return (rows_v.at[pl.ds(slot * CHUNK, CHUNK)],
                        bank_ref.at[dst2.at[c]], ssem)

            def when_chunk(pred, fn):
                @pl.when(pred)
                def _():
                    fn()

            for c0 in range(PIPE):
                when_chunk(c0 < n_chunks,
                           lambda c=c0: pltpu.async_copy(*g_args(jnp.int32(c))))

            @pl.loop(0, n_chunks)
            def _(c):
                when_chunk(c >= PIPE,
                           lambda: pltpu.make_async_copy(*s_args(c - PIPE)).wait())
                pltpu.make_async_copy(*g_args(c)).wait()
                pltpu.async_copy(*s_args(c))
                when_chunk(c + PIPE < n_chunks,
                           lambda: pltpu.async_copy(*g_args(c + PIPE)))

            @pl.loop(lax.max(jnp.int32(0), n_chunks - PIPE), n_chunks)
            def _(c):
                pltpu.make_async_copy(*s_args(c)).wait()

    return scatter_kernel


def _gru_pallas(x, h, w_ihT, w_hhT, b_ih2, b_hh2, bank_in):
    B, D = x.shape
    H = h.shape[1]
    N = bank_in.shape[0]
    R = 2048

    def body(x_ref, h_ref, wi_ref, wh_ref, bi_ref, bh_ref, bank_ref,
             out_ref, bank_out_ref):
        del bank_ref, bank_out_ref
        xb = x_ref[...]
        hb = h_ref[...]
        gi = jnp.dot(xb, wi_ref[...], preferred_element_type=jnp.float32) + bi_ref[...]
        gh = jnp.dot(hb, wh_ref[...], preferred_element_type=jnp.float32) + bh_ref[...]
        r = jax.nn.sigmoid(gi[:, :H] + gh[:, :H])
        z = jax.nn.sigmoid(gi[:, H:2 * H] + gh[:, H:2 * H])
        n = jnp.tanh(gi[:, 2 * H:] + r * gh[:, 2 * H:])
        out_ref[...] = (1.0 - z) * n + z * hb

    return pl.pallas_call(
        body,
        grid=(B // R,),
        in_specs=[
            pl.BlockSpec((R, D), lambda i: (i, 0)),
            pl.BlockSpec((R, H), lambda i: (i, 0)),
            pl.BlockSpec((D, 3 * H), lambda i: (0, 0)),
            pl.BlockSpec((H, 3 * H), lambda i: (0, 0)),
            pl.BlockSpec((1, 3 * H), lambda i: (0, 0)),
            pl.BlockSpec((1, 3 * H), lambda i: (0, 0)),
            pl.BlockSpec(memory_space=pl.ANY),
        ],
        out_specs=[pl.BlockSpec((R, H), lambda i: (i, 0)),
                   pl.BlockSpec(memory_space=pl.ANY)],
        out_shape=[jax.ShapeDtypeStruct((B, H), jnp.float32),
                   jax.ShapeDtypeStruct((N, H), jnp.float32)],
        input_output_aliases={6: 1},
    )(x, h, w_ihT, w_hhT, b_ih2, b_hh2, bank_in)


def kernel(current_time, mgn_h, delta_t, X_obs, M_obs, i_obs, update,
           W_ih, W_hh, b_ih, b_hh):
    N, H = mgn_h.shape
    B, D = X_obs.shape
    ii = i_obs.astype(jnp.int32)

    h_obs, dst_plan, src_plan, cnt_plan = _make_gather_plan(N, B, H)(mgn_h, ii)
    h_new, bank_out = _gru_pallas(X_obs, h_obs, W_ih.T, W_hh.T,
                                  b_ih.reshape(1, -1), b_hh.reshape(1, -1),
                                  mgn_h)
    bank = jax.new_ref(bank_out)
    _make_scatter(N, B, H)(bank, h_new, dst_plan, src_plan, cnt_plan)
    return bank[...]


# speedup vs baseline: 1.6981x; 1.0156x over previous
import dataclasses
import functools

import jax
import jax.numpy as jnp
from jax import lax
from jax.experimental import pallas as pl
from jax.experimental.pallas import tpu as pltpu
from jax.experimental.pallas import tpu_sc as plsc

NC = 2
NS = 16
NW = NC * NS
CHUNK = 128
L = 16
PIPE = 3
NSLOT = 2 * PIPE

_INT_MIN = -2147483648


def _sc_mesh():
    return plsc.VectorSubcoreMesh(core_axis_name="c", subcore_axis_name="s")


def _sc_cp():
    cp = pltpu.CompilerParams()
    if "needs_layout_passes" in pltpu.CompilerParams.__dataclass_fields__:
        cp = dataclasses.replace(cp, needs_layout_passes=False)
    return cp


def _lane0(v):
    iota = lax.iota(jnp.int32, L)
    return lax.reduce_max(
        jnp.where(iota == 0, v, jnp.full((L,), _INT_MIN, jnp.int32)), (0,))


@functools.lru_cache(maxsize=None)
def _make_gather_plan(N, B, D):
    b_per_w = B // NW
    n_gch = b_per_w // CHUNK
    rows_per_w = N // NW
    win_sz = ((rows_per_w + L - 1) // L) * L
    n_win_vecs = win_sz // L
    max_ch = (rows_per_w + CHUNK - 1) // CHUNK + 1
    list_sz = max_ch * CHUNK

    @functools.partial(
        pl.kernel,
        out_type=[jax.ShapeDtypeStruct((B, D), jnp.float32),
                  jax.ShapeDtypeStruct((NW, max_ch, CHUNK), jnp.int32),
                  jax.ShapeDtypeStruct((NW, max_ch, CHUNK), jnp.int32),
                  jax.ShapeDtypeStruct((NW, L), jnp.int32)],
        mesh=_sc_mesh(),
        compiler_params=_sc_cp(),
        scratch_types=[
            pltpu.VMEM((B,), jnp.int32),
            pltpu.VMEM((b_per_w, D), jnp.float32),
            pltpu.VMEM((win_sz,), jnp.int32),
            pltpu.VMEM((list_sz,), jnp.int32),
            pltpu.VMEM((list_sz,), jnp.int32),
            pltpu.VMEM((max_ch, CHUNK), jnp.int32),
            pltpu.VMEM((max_ch, CHUNK), jnp.int32),
            pltpu.VMEM((L,), jnp.int32),
            pltpu.SemaphoreType.DMA,
        ],
    )
    def gather_plan_kernel(bank_hbm, idx_hbm, hobs_hbm, dst_hbm, src_hbm,
                           cnt_hbm, idx_all, rows_v, win_v, dstf, srcf,
                           dst2, src2, cnt_v, sem):
        wid = lax.axis_index("s") * NC + lax.axis_index("c")
        lo = wid * rows_per_w
        hi = lo + rows_per_w
        iota = lax.iota(jnp.int32, L)
        pltpu.sync_copy(idx_hbm, idx_all)

        def g_args(j):
            return (bank_hbm.at[idx_all.at[pl.ds(wid * b_per_w + j * CHUNK, CHUNK)]],
                    rows_v.at[pl.ds(j * CHUNK, CHUNK)], sem)

        for j in range(n_gch):
            pltpu.async_copy(*g_args(j))

        neg1 = jnp.full((L,), -1, jnp.int32)

        @pl.loop(0, n_win_vecs)
        def _(j):
            win_v[pl.ds(j * L, L)] = neg1

        @pl.loop(0, B // (4 * L))
        def _(i):
            for u in range(4):
                base = (4 * i + u) * L
                v = idx_all[pl.ds(base, L)]
                b = iota + base
                m = (v >= lo) & (v < hi)
                plsc.store_scatter(win_v.at[:], [v - lo], b, mask=m)

        def compact(j, off):
            wv = win_v[pl.ds(j * L, L)]
            m = wv >= 0
            rows = iota + (j * L + lo)
            plsc.store_compressed(dstf.at[pl.ds(off, L)], rows, mask=m)
            plsc.store_compressed(srcf.at[pl.ds(off, L)], wv, mask=m)
            return off + lax.reduce_max(plsc.all_reduce_population_count(m), (0,))

        cnt = lax.fori_loop(0, n_win_vecs, compact, jnp.int32(0))
        n_chunks = (cnt + CHUNK - 1) // CHUNK
        cnt_v[...] = jnp.full((L,), 0, jnp.int32) + n_chunks
        pltpu.sync_copy(cnt_v, cnt_hbm.at[wid])

        @pl.when(cnt > 0)
        def _():
            d0 = _lane0(dstf[pl.ds(0, L)])
            s0 = _lane0(srcf[pl.ds(0, L)])

            @pl.loop(0, (n_chunks * CHUNK - (cnt // L) * L) // L)
            def _(t):
                base = (cnt // L) * L + t * L
                keep = iota + base < cnt
                dv = dstf[pl.ds(base, L)]
                sv = srcf[pl.ds(base, L)]
                dstf[pl.ds(base, L)] = jnp.where(keep, dv, jnp.full((L,), 0, jnp.int32) + d0)
                srcf[pl.ds(base, L)] = jnp.where(keep, sv, jnp.full((L,), 0, jnp.int32) + s0)

            @pl.loop(0, n_chunks * (CHUNK // L))
            def _(k):
                c = k // (CHUNK // L)
                kk = lax.rem(k, jnp.int32(CHUNK // L))
                dst2[c, pl.ds(kk * L, L)] = dstf[pl.ds(k * L, L)]
                src2[c, pl.ds(kk * L, L)] = srcf[pl.ds(k * L, L)]

            pltpu.sync_copy(dst2, dst_hbm.at[wid])
            pltpu.sync_copy(src2, src_hbm.at[wid])

        for j in range(n_gch):
            pltpu.make_async_copy(*g_args(j)).wait()
        pltpu.sync_copy(rows_v, hobs_hbm.at[pl.ds(wid * b_per_w, b_per_w)])

    return gather_plan_kernel


@functools.lru_cache(maxsize=None)
def _make_scatter(N, B, D):
    rows_per_w = N // NW
    max_ch = (rows_per_w + CHUNK - 1) // CHUNK + 1

    @functools.partial(
        pl.kernel,
        out_type=(),
        mesh=_sc_mesh(),
        compiler_params=_sc_cp(),
        scratch_types=[
            pltpu.VMEM((max_ch, CHUNK), jnp.int32),
            pltpu.VMEM((max_ch, CHUNK), jnp.int32),
            pltpu.VMEM((L,), jnp.int32),
            pltpu.VMEM((NSLOT * CHUNK, D), jnp.float32),
            pltpu.SemaphoreType.DMA,
            pltpu.SemaphoreType.DMA,
        ],
    )
    def scatter_kernel(bank_ref, hnew_hbm, dst_hbm, src_hbm, cnt_hbm,
                       dst2, src2, cnt_v, rows_v, gsem, ssem):
        wid = lax.axis_index("s") * NC + lax.axis_index("c")
        pltpu.sync_copy(cnt_hbm.at[wid], cnt_v)
        n_chunks = _lane0(cnt_v[...])

        @pl.when(n_chunks > 0)
        def _():
            pltpu.sync_copy(dst_hbm.at[wid], dst2)
            pltpu.sync_copy(src_hbm.at[wid], src2)

            def g_args(c):
                slot = lax.rem(c, jnp.int32(NSLOT))
                return (hnew_hbm.at[src2.at[c]],
                        rows_v.at[pl.ds(slot * CHUNK, CHUNK)], gsem)

            def s_args(c):
                slot = lax.rem(c, jnp.int32(NSLOT))
                return (rows_v.at[pl.ds(slot * CHUNK, CHUNK)],
                        bank_ref.at[dst2.at[c]], ssem)

            def when_chunk(pred, fn):
                @pl.when(pred)
                def _():
                    fn()

            for c0 in range(PIPE):
                when_chunk(c0 < n_chunks,
                           lambda c=c0: pltpu.async_copy(*g_args(jnp.int32(c))))

            @pl.loop(0, n_chunks)
            def _(c):
                when_chunk(c >= PIPE,
                           lambda: pltpu.make_async_copy(*s_args(c - PIPE)).wait())
                pltpu.make_async_copy(*g_args(c)).wait()
                pltpu.async_copy(*s_args(c))
                when_chunk(c + PIPE < n_chunks,
                           lambda: pltpu.async_copy(*g_args(c + PIPE)))

            @pl.loop(lax.max(jnp.int32(0), n_chunks - PIPE), n_chunks)
            def _(c):
                pltpu.make_async_copy(*s_args(c)).wait()

    return scatter_kernel


def _gru_pallas(x, h, w_ihT, w_hhT, b_ih2, b_hh2, bank_in):
    B, D = x.shape
    H = h.shape[1]
    N = bank_in.shape[0]
    R = 4096

    def body(x_ref, h_ref, wi_ref, wh_ref, bi_ref, bh_ref, bank_ref,
             out_ref, bank_out_ref):
        del bank_ref, bank_out_ref
        xb = x_ref[...]
        hb = h_ref[...]
        gi = jnp.dot(xb, wi_ref[...], preferred_element_type=jnp.float32) + bi_ref[...]
        gh = jnp.dot(hb, wh_ref[...], preferred_element_type=jnp.float32) + bh_ref[...]
        r = jax.nn.sigmoid(gi[:, :H] + gh[:, :H])
        z = jax.nn.sigmoid(gi[:, H:2 * H] + gh[:, H:2 * H])
        n = jnp.tanh(gi[:, 2 * H:] + r * gh[:, 2 * H:])
        out_ref[...] = (1.0 - z) * n + z * hb

    return pl.pallas_call(
        body,
        grid=(B // R,),
        in_specs=[
            pl.BlockSpec((R, D), lambda i: (i, 0)),
            pl.BlockSpec((R, H), lambda i: (i, 0)),
            pl.BlockSpec((D, 3 * H), lambda i: (0, 0)),
            pl.BlockSpec((H, 3 * H), lambda i: (0, 0)),
            pl.BlockSpec((1, 3 * H), lambda i: (0, 0)),
            pl.BlockSpec((1, 3 * H), lambda i: (0, 0)),
            pl.BlockSpec(memory_space=pl.ANY),
        ],
        out_specs=[pl.BlockSpec((R, H), lambda i: (i, 0)),
                   pl.BlockSpec(memory_space=pl.ANY)],
        out_shape=[jax.ShapeDtypeStruct((B, H), jnp.float32),
                   jax.ShapeDtypeStruct((N, H), jnp.float32)],
        input_output_aliases={6: 1},
    )(x, h, w_ihT, w_hhT, b_ih2, b_hh2, bank_in)


def kernel(current_time, mgn_h, delta_t, X_obs, M_obs, i_obs, update,
           W_ih, W_hh, b_ih, b_hh):
    N, H = mgn_h.shape
    B, D = X_obs.shape
    ii = i_obs.astype(jnp.int32)

    h_obs, dst_plan, src_plan, cnt_plan = _make_gather_plan(N, B, H)(mgn_h, ii)
    h_new, bank_out = _gru_pallas(X_obs, h_obs, W_ih.T, W_hh.T,
                                  b_ih.reshape(1, -1), b_hh.reshape(1, -1),
                                  mgn_h)
    bank = jax.new_ref(bank_out)
    _make_scatter(N, B, H)(bank, h_new, dst_plan, src_plan, cnt_plan)
    return bank[...]
